# baseline (speedup 1.0000x reference)
import functools

import jax
import jax.numpy as jnp
from jax.experimental import pallas as pl
from jax.experimental.pallas import tpu as pltpu


def _start_block_copy(x_hbm, xbuf, sems, row, slot, tb, nsplit):
    sub = tb // nsplit
    for k in range(nsplit):
        pltpu.make_async_copy(
            x_hbm.at[pl.ds(row + k * sub, sub), 0, :],
            xbuf.at[slot, pl.ds(k * sub, sub), :],
            sems.at[slot, k]).start()


def _wait_block_copy(xbuf, sems, slot, tb, nsplit):
    sub = tb // nsplit
    for k in range(nsplit):
        pltpu.make_async_copy(
            xbuf.at[slot, pl.ds(k * sub, sub), :],
            xbuf.at[slot, pl.ds(k * sub, sub), :],
            sems.at[slot, k]).wait()


def _cls_dense_tanh_kernel(x_hbm, w_ref, b_ref, o_ref, xbuf, sems, *,
                           tb, nb, nsplit):
    i = pl.program_id(0)
    j = pl.program_id(1)

    @pl.when(j == 0)
    def _():
        row0 = i * nb * tb
        _start_block_copy(x_hbm, xbuf, sems, row0, 0, tb, nsplit)

    @pl.when(j + 1 < nb)
    def _():
        row = (i * nb + j + 1) * tb
        _start_block_copy(x_hbm, xbuf, sems, row, (j + 1) % 2, tb, nsplit)

    slot = j % 2
    bias = b_ref[...].reshape(1, -1)
    sub = tb // nsplit
    for k in range(nsplit):
        pltpu.make_async_copy(
            xbuf.at[slot, pl.ds(k * sub, sub), :],
            xbuf.at[slot, pl.ds(k * sub, sub), :],
            sems.at[slot, k]).wait()
        y = jnp.dot(xbuf[slot, pl.ds(k * sub, sub), :], w_ref[...],
                    preferred_element_type=jnp.float32)
        o_ref[pl.ds(k * sub, sub), :] = jnp.tanh(y + bias).astype(o_ref.dtype)


def kernel(x, w, b):
    B, S, H = x.shape
    O = w.shape[1]
    o_pad = max(128, ((O + 127) // 128) * 128)
    if o_pad != O:
        w = jnp.pad(w, ((0, 0), (0, o_pad - O)))
        b = jnp.pad(b, (0, o_pad - O))

    tb = 512
    while B % (2 * tb) != 0:
        tb //= 2
    nb = B // (2 * tb)
    nsplit = 4
    while tb % nsplit != 0:
        nsplit //= 2

    out = pl.pallas_call(
        functools.partial(_cls_dense_tanh_kernel, tb=tb, nb=nb,
                          nsplit=nsplit),
        out_shape=jax.ShapeDtypeStruct((B, o_pad), x.dtype),
        grid=(2, nb),
        in_specs=[
            pl.BlockSpec(memory_space=pl.ANY),
            pl.BlockSpec((H, o_pad), lambda i, j: (0, 0)),
            pl.BlockSpec((o_pad,), lambda i, j: (0,)),
        ],
        out_specs=pl.BlockSpec((tb, o_pad), lambda i, j: (i * nb + j, 0)),
        scratch_shapes=[
            pltpu.VMEM((2, tb, H), x.dtype),
            pltpu.SemaphoreType.DMA((2, nsplit)),
        ],
        compiler_params=pltpu.CompilerParams(
            dimension_semantics=("parallel", "arbitrary"),
            vmem_limit_bytes=64 * 1024 * 1024,
        ),
        cost_estimate=pl.CostEstimate(
            flops=2 * B * H * o_pad,
            transcendentals=B * o_pad,
            bytes_accessed=(B * H * 4 + H * o_pad * 4 + B * o_pad * 4),
        ),
    )(x, w, b)

    if o_pad != O:
        out = out[:, :O]
    return out


# speedup vs baseline: 1.2153x; 1.2153x over previous
import functools

import jax
import jax.numpy as jnp
from jax.experimental import pallas as pl
from jax.experimental.pallas import tpu as pltpu


def _start_block_copy(x_hbm, xbuf, sems, row, slot, tb, nsplit):
    sub = tb // nsplit
    for k in range(nsplit):
        pltpu.make_async_copy(
            x_hbm.at[pl.ds(row + k * sub, sub), 0, :],
            xbuf.at[slot, pl.ds(k * sub, sub), :],
            sems.at[slot, k]).start()


def _wait_block_copy(xbuf, sems, slot, tb, nsplit):
    sub = tb // nsplit
    for k in range(nsplit):
        pltpu.make_async_copy(
            xbuf.at[slot, pl.ds(k * sub, sub), :],
            xbuf.at[slot, pl.ds(k * sub, sub), :],
            sems.at[slot, k]).wait()


def _cls_dense_tanh_kernel(x_hbm, w_hbm, b_ref, o_ref, xbuf, wvmem, sems,
                           wsem, *, tb, nb, nsplit):
    i = pl.program_id(0)
    j = pl.program_id(1)

    @pl.when(j == 0)
    def _():
        row0 = i * nb * tb
        _start_block_copy(x_hbm, xbuf, sems, row0, 0, tb, nsplit)
        pltpu.make_async_copy(w_hbm, wvmem, wsem).start()

    @pl.when(j + 1 < nb)
    def _():
        row = (i * nb + j + 1) * tb
        _start_block_copy(x_hbm, xbuf, sems, row, (j + 1) % 2, tb, nsplit)

    @pl.when(j == 0)
    def _():
        pltpu.make_async_copy(wvmem, wvmem, wsem).wait()

    slot = j % 2
    _wait_block_copy(xbuf, sems, slot, tb, nsplit)
    y = jnp.dot(xbuf[slot], wvmem[...], preferred_element_type=jnp.float32)
    y = y + b_ref[...].reshape(1, -1)
    o_ref[...] = jnp.tanh(y).astype(o_ref.dtype)


def kernel(x, w, b):
    B, S, H = x.shape
    O = w.shape[1]
    o_pad = max(128, ((O + 127) // 128) * 128)
    if o_pad != O:
        w = jnp.pad(w, ((0, 0), (0, o_pad - O)))
        b = jnp.pad(b, (0, o_pad - O))

    tb = 512
    while B % (2 * tb) != 0:
        tb //= 2
    nb = B // (2 * tb)
    nsplit = 4
    while tb % nsplit != 0:
        nsplit //= 2

    out = pl.pallas_call(
        functools.partial(_cls_dense_tanh_kernel, tb=tb, nb=nb,
                          nsplit=nsplit),
        out_shape=jax.ShapeDtypeStruct((B, o_pad), x.dtype),
        grid=(2, nb),
        in_specs=[
            pl.BlockSpec(memory_space=pl.ANY),
            pl.BlockSpec(memory_space=pl.ANY),
            pl.BlockSpec((o_pad,), lambda i, j: (0,)),
        ],
        out_specs=pl.BlockSpec((tb, o_pad), lambda i, j: (i * nb + j, 0)),
        scratch_shapes=[
            pltpu.VMEM((2, tb, H), x.dtype),
            pltpu.VMEM((H, o_pad), jnp.float32),
            pltpu.SemaphoreType.DMA((2, nsplit)),
            pltpu.SemaphoreType.DMA,
        ],
        compiler_params=pltpu.CompilerParams(
            dimension_semantics=("parallel", "arbitrary"),
            vmem_limit_bytes=64 * 1024 * 1024,
        ),
        cost_estimate=pl.CostEstimate(
            flops=2 * B * H * o_pad,
            transcendentals=B * o_pad,
            bytes_accessed=(B * H * 4 + H * o_pad * 4 + B * o_pad * 4),
        ),
    )(x, w, b)

    if o_pad != O:
        out = out[:, :O]
    return out


# speedup vs baseline: 1.2975x; 1.0676x over previous
import functools

import jax
import jax.numpy as jnp
from jax.experimental import pallas as pl
from jax.experimental.pallas import tpu as pltpu


def _start_block_copy(x_hbm, xbuf, sems, row, slot, tb, nsplit):
    sub = tb // nsplit
    for k in range(nsplit):
        pltpu.make_async_copy(
            x_hbm.at[pl.ds(row + k * sub, sub), 0, :],
            xbuf.at[slot, pl.ds(k * sub, sub), :],
            sems.at[slot, k]).start()


def _wait_block_copy(xbuf, sems, slot, tb, nsplit):
    sub = tb // nsplit
    for k in range(nsplit):
        pltpu.make_async_copy(
            xbuf.at[slot, pl.ds(k * sub, sub), :],
            xbuf.at[slot, pl.ds(k * sub, sub), :],
            sems.at[slot, k]).wait()


def _cls_dense_tanh_kernel(x_hbm, w_ref, b_ref, o_ref, xbuf, sems, *,
                           tb, nb, nsplit):
    i = pl.program_id(0)
    j = pl.program_id(1)

    @pl.when(j == 0)
    def _():
        row0 = i * nb * tb
        _start_block_copy(x_hbm, xbuf, sems, row0, 0, tb, nsplit)

    @pl.when(j + 1 < nb)
    def _():
        row = (i * nb + j + 1) * tb
        _start_block_copy(x_hbm, xbuf, sems, row, (j + 1) % 2, tb, nsplit)

    slot = j % 2
    _wait_block_copy(xbuf, sems, slot, tb, nsplit)
    y = jnp.dot(xbuf[slot], w_ref[...], preferred_element_type=jnp.float32)
    y = y + b_ref[...].reshape(1, -1)
    o_ref[...] = jnp.tanh(y).astype(o_ref.dtype)


def kernel(x, w, b):
    B, S, H = x.shape
    O = w.shape[1]
    o_pad = max(128, ((O + 127) // 128) * 128)
    if o_pad != O:
        w = jnp.pad(w, ((0, 0), (0, o_pad - O)))
        b = jnp.pad(b, (0, o_pad - O))

    tb = 512
    while B % (2 * tb) != 0:
        tb //= 2
    nb = B // (2 * tb)
    nsplit = 1

    out = pl.pallas_call(
        functools.partial(_cls_dense_tanh_kernel, tb=tb, nb=nb,
                          nsplit=nsplit),
        out_shape=jax.ShapeDtypeStruct((B, o_pad), x.dtype),
        grid=(2, nb),
        in_specs=[
            pl.BlockSpec(memory_space=pl.ANY),
            pl.BlockSpec((H, o_pad), lambda i, j: (0, 0)),
            pl.BlockSpec((o_pad,), lambda i, j: (0,)),
        ],
        out_specs=pl.BlockSpec((tb, o_pad), lambda i, j: (i * nb + j, 0)),
        scratch_shapes=[
            pltpu.VMEM((2, tb, H), x.dtype),
            pltpu.SemaphoreType.DMA((2, nsplit)),
        ],
        compiler_params=pltpu.CompilerParams(
            dimension_semantics=("parallel", "arbitrary"),
            vmem_limit_bytes=64 * 1024 * 1024,
        ),
        cost_estimate=pl.CostEstimate(
            flops=2 * B * H * o_pad,
            transcendentals=B * o_pad,
            bytes_accessed=(B * H * 4 + H * o_pad * 4 + B * o_pad * 4),
        ),
    )(x, w, b)

    if o_pad != O:
        out = out[:, :O]
    return out


# speedup vs baseline: 1.3114x; 1.0107x over previous
import functools

import jax
import jax.numpy as jnp
from jax.experimental import pallas as pl
from jax.experimental.pallas import tpu as pltpu


def _start_block_copy(x_hbm, xbuf, sems, row, slot, tb):
    pltpu.make_async_copy(
        x_hbm.at[pl.ds(row, tb), 0, :], xbuf.at[slot], sems.at[slot]).start()


def _cls_dense_tanh_kernel(x_hbm, w_ref, b_ref, o_ref, xbuf, sems, *, tb, nb):
    i = pl.program_id(0)
    j = pl.program_id(1)

    @pl.when(j == 0)
    def _():
        _start_block_copy(x_hbm, xbuf, sems, i * nb * tb, 0, tb)

    @pl.when(j + 1 < nb)
    def _():
        row = (i * nb + j + 1) * tb
        _start_block_copy(x_hbm, xbuf, sems, row, (j + 1) % 2, tb)

    slot = j % 2
    pltpu.make_async_copy(xbuf.at[slot], xbuf.at[slot], sems.at[slot]).wait()
    y = jnp.dot(xbuf[slot], w_ref[...], preferred_element_type=jnp.float32)
    y = y + b_ref[...].reshape(1, -1)
    o_ref[...] = jnp.tanh(y).astype(o_ref.dtype)


def kernel(x, w, b):
    B, S, H = x.shape
    O = w.shape[1]
    o_pad = max(128, ((O + 127) // 128) * 128)
    if o_pad != O:
        w = jnp.pad(w, ((0, 0), (0, o_pad - O)))
        b = jnp.pad(b, (0, o_pad - O))

    tb = 512
    while B % (2 * tb) != 0:
        tb //= 2
    nb = B // (2 * tb)

    out = pl.pallas_call(
        functools.partial(_cls_dense_tanh_kernel, tb=tb, nb=nb),
        out_shape=jax.ShapeDtypeStruct((B, o_pad), x.dtype),
        grid=(2, nb),
        in_specs=[
            pl.BlockSpec(memory_space=pl.ANY),
            pl.BlockSpec((H, o_pad), lambda i, j: (0, 0)),
            pl.BlockSpec((o_pad,), lambda i, j: (0,)),
        ],
        out_specs=pl.BlockSpec((tb, o_pad), lambda i, j: (i * nb + j, 0)),
        scratch_shapes=[
            pltpu.VMEM((2, tb, H), x.dtype),
            pltpu.SemaphoreType.DMA((2,)),
        ],
        compiler_params=pltpu.CompilerParams(
            dimension_semantics=("parallel", "arbitrary"),
            vmem_limit_bytes=64 * 1024 * 1024,
        ),
        cost_estimate=pl.CostEstimate(
            flops=2 * B * H * o_pad,
            transcendentals=B * o_pad,
            bytes_accessed=(B * H * 4 + H * o_pad * 4 + B * o_pad * 4),
        ),
    )(x, w, b)

    if o_pad != O:
        out = out[:, :O]
    return out
